# baseline (speedup 1.0000x reference)
import functools

import jax
import jax.numpy as jnp
from jax import lax
from jax.experimental import pallas as pl
from jax.experimental.pallas import tpu as pltpu
from jax.experimental.pallas import tpu_sc as plsc

BATCH = 16384
EMBED = 32
HIDDEN = 256

_NC, _NS = 2, 16
_NW = _NC * _NS
_B_PER_W = BATCH // _NW
_CHUNK = 128
_NCHUNK = _B_PER_W // _CHUNK


def _sc_gather(user_id2, movie_id2, user_table, movie_table):
    mesh = plsc.VectorSubcoreMesh(core_axis_name="c", subcore_axis_name="s")

    @functools.partial(
        pl.kernel,
        mesh=mesh,
        out_type=[
            jax.ShapeDtypeStruct((BATCH, EMBED), jnp.float32),
            jax.ShapeDtypeStruct((BATCH, EMBED), jnp.float32),
        ],
        scratch_types=[
            pltpu.VMEM((_NCHUNK, _CHUNK), jnp.int32),
            pltpu.VMEM((_NCHUNK, _CHUNK), jnp.int32),
            pltpu.VMEM((_B_PER_W, EMBED), jnp.float32),
            pltpu.VMEM((_B_PER_W, EMBED), jnp.float32),
            pltpu.SemaphoreType.DMA,
        ],
        compiler_params=pltpu.CompilerParams(use_tc_tiling_on_sc=False),
    )
    def k(uid_hbm, mid_hbm, utab_hbm, mtab_hbm, uout_hbm, mout_hbm,
          uidx_v, midx_v, urows_v, mrows_v, sem):
        wid = lax.axis_index("s") * _NC + lax.axis_index("c")
        base = wid * _B_PER_W
        pltpu.sync_copy(uid_hbm.at[wid], uidx_v)
        pltpu.sync_copy(mid_hbm.at[wid], midx_v)
        copies = []
        for c in range(_NCHUNK):
            copies.append(pltpu.async_copy(
                utab_hbm.at[uidx_v.at[c]],
                urows_v.at[pl.ds(c * _CHUNK, _CHUNK)], sem))
            copies.append(pltpu.async_copy(
                mtab_hbm.at[midx_v.at[c]],
                mrows_v.at[pl.ds(c * _CHUNK, _CHUNK)], sem))
        for cp in copies:
            cp.wait()
        pltpu.sync_copy(urows_v, uout_hbm.at[pl.ds(base, _B_PER_W)])
        pltpu.sync_copy(mrows_v, mout_hbm.at[pl.ds(base, _B_PER_W)])

    return k(user_id2, movie_id2, user_table, movie_table)


def _mlp_body(u_ref, m_ref, w1u_ref, w1m_ref, b1_ref, w2_ref, b2_ref, o_ref):
    x = (jnp.dot(u_ref[...], w1u_ref[...], preferred_element_type=jnp.float32)
         + jnp.dot(m_ref[...], w1m_ref[...], preferred_element_type=jnp.float32)
         + b1_ref[...])
    h = jnp.maximum(x, 0.0)
    o_ref[...] = (jnp.dot(h, w2_ref[...], preferred_element_type=jnp.float32)
                  + b2_ref[...])


def _tc_mlp(u_emb, m_emb, W1u, W1m, b1, W2, b2, block_m=2048):
    grid = (BATCH // block_m,)
    return pl.pallas_call(
        _mlp_body,
        grid=grid,
        in_specs=[
            pl.BlockSpec((block_m, EMBED), lambda i: (i, 0)),
            pl.BlockSpec((block_m, EMBED), lambda i: (i, 0)),
            pl.BlockSpec((EMBED, HIDDEN), lambda i: (0, 0)),
            pl.BlockSpec((EMBED, HIDDEN), lambda i: (0, 0)),
            pl.BlockSpec((1, HIDDEN), lambda i: (0, 0)),
            pl.BlockSpec((HIDDEN, 1), lambda i: (0, 0)),
            pl.BlockSpec((1, 1), lambda i: (0, 0)),
        ],
        out_specs=pl.BlockSpec((block_m, 1), lambda i: (i, 0)),
        out_shape=jax.ShapeDtypeStruct((BATCH, 1), jnp.float32),
    )(u_emb, m_emb, W1u, W1m, b1, W2, b2)


def kernel(user_id, movie_title, user_table, movie_table, W1, b1, W2, b2):
    uid2 = user_id.astype(jnp.int32).reshape(_NW, _NCHUNK, _CHUNK)
    mid2 = movie_title.astype(jnp.int32).reshape(_NW, _NCHUNK, _CHUNK)
    u_emb, m_emb = _sc_gather(uid2, mid2, user_table, movie_table)
    W1u = W1[:EMBED]
    W1m = W1[EMBED:]
    return _tc_mlp(u_emb, m_emb, W1u, W1m,
                   b1.reshape(1, HIDDEN), W2, b2.reshape(1, 1))
